# Initial kernel scaffold; baseline (speedup 1.0000x reference)
import jax
import jax.numpy as jnp
from jax.experimental import pallas as pl


def kernel(atom_features, Wself0, Wrel0, bias0, Wself1, Wrel1, bias1, Wself2, Wrel2, bias2, Wself3, Wrel3, bias3, W1, b1, gamma, beta, W2, b2, degree_slice, membership, deg_adj_1, deg_adj_2, deg_adj_3, deg_adj_4):
    raise NotImplementedError("write your pallas kernel here")


# speedup vs baseline: 1.1319x; 1.1319x over previous
import functools
import math

import jax
import jax.numpy as jnp
from jax import lax
from jax.experimental import pallas as pl
from jax.experimental.pallas import tpu as pltpu
from jax.experimental.pallas import tpu_sc as plsc

N_PER_DEG = 20000
N_ATOMS = 5 * N_PER_DEG
BATCH = 512
NTASKS = 12
PADS = [16, 32, 32, 48]
NC, NS = 2, 16
NW = NC * NS
CH = 160
NCH = N_PER_DEG // CH

_SELU_ALPHA = 1.6732632423543772
_SELU_SCALE = 1.0507009873554805


def _selu(x):
    return _SELU_SCALE * jnp.where(x > 0, x, _SELU_ALPHA * (jnp.exp(x) - 1.0))


def _layer_mm_body(x_ref, wself_ref, wrel_ref, bias_ref, s_ref, p_ref):
    x = x_ref[...]
    s_ref[...] = jnp.dot(x, wself_ref[0], preferred_element_type=jnp.float32) + bias_ref[0]
    for d in range(4):
        p_ref[d] = jnp.dot(x, wrel_ref[d], preferred_element_type=jnp.float32)


def _layer_matmuls(x, wself, wrel, bias3d, din_p, dp):
    bs = 2000
    nblk = N_ATOMS // bs
    per_bucket = N_PER_DEG // bs
    return pl.pallas_call(
        _layer_mm_body,
        grid=(nblk,),
        in_specs=[
            pl.BlockSpec((bs, din_p), lambda i: (i, 0)),
            pl.BlockSpec((1, din_p, dp), lambda i: (i // per_bucket, 0, 0)),
            pl.BlockSpec((4, din_p, dp), lambda i: (0, 0, 0)),
            pl.BlockSpec((1, 1, dp), lambda i: (i // per_bucket, 0, 0)),
        ],
        out_specs=[
            pl.BlockSpec((bs, dp), lambda i: (i, 0)),
            pl.BlockSpec((4, bs, dp), lambda i: (0, i, 0)),
        ],
        out_shape=[
            jax.ShapeDtypeStruct((N_ATOMS, dp), jnp.float32),
            jax.ShapeDtypeStruct((4, N_ATOMS, dp), jnp.float32),
        ],
    )(x, wself, wrel, bias3d)


def _make_conv_sc(dp):
    nv = dp // 16
    mesh = plsc.VectorSubcoreMesh(core_axis_name="c", subcore_axis_name="s")

    @functools.partial(
        pl.kernel,
        mesh=mesh,
        out_type=jax.ShapeDtypeStruct((N_ATOMS, dp), jnp.float32),
        compiler_params=pltpu.CompilerParams(use_tc_tiling_on_sc=False),
        scratch_types=[
            pltpu.VMEM((CH, dp), jnp.float32),
            pltpu.VMEM((4 * CH,), jnp.int32),
            pltpu.VMEM((4 * CH, dp), jnp.float32),
            pltpu.VMEM((CH, dp), jnp.float32),
            pltpu.SemaphoreType.DMA,
        ],
    )
    def conv_sc(s_hbm, p1, p2, p3, p4, a1, a2, a3, a4, out_hbm,
                sbuf, idxbuf, rowsbuf, outbuf, sem):
        wid = lax.axis_index("s") * NC + lax.axis_index("c")
        ptabs = [p1, p2, p3, p4]
        adjs = [a1, a2, a3, a4]
        for k in range(4):
            cid = wid + NW * k

            @pl.when(cid < NCH)
            def _():
                for deg in range(5):
                    base = deg * N_PER_DEG + cid * CH
                    pltpu.sync_copy(s_hbm.at[pl.ds(base, CH)], sbuf)
                    if deg > 0:
                        n = CH * deg
                        pltpu.sync_copy(adjs[deg - 1].at[pl.ds(cid * n, n)],
                                        idxbuf.at[pl.ds(0, n)])
                        pltpu.async_copy(ptabs[deg - 1].at[idxbuf.at[pl.ds(0, n)]],
                                         rowsbuf.at[pl.ds(0, n)], sem).wait()

                    def row_body(i, _, deg=deg):
                        for v in range(nv):
                            acc = sbuf[i, pl.ds(v * 16, 16)]
                            for j in range(deg):
                                acc = acc + rowsbuf[i * deg + j, pl.ds(v * 16, 16)]
                            outbuf[i, pl.ds(v * 16, 16)] = _selu(acc)
                        return 0

                    lax.fori_loop(0, CH, row_body, 0)
                    pltpu.sync_copy(outbuf, out_hbm.at[pl.ds(base, CH)])

    return conv_sc


def _make_pool_sc(dp):
    nv = dp // 16
    mesh = plsc.VectorSubcoreMesh(core_axis_name="c", subcore_axis_name="s")

    @functools.partial(
        pl.kernel,
        mesh=mesh,
        out_type=jax.ShapeDtypeStruct((N_ATOMS, dp), jnp.float32),
        compiler_params=pltpu.CompilerParams(use_tc_tiling_on_sc=False),
        scratch_types=[
            pltpu.VMEM((CH, dp), jnp.float32),
            pltpu.VMEM((4 * CH,), jnp.int32),
            pltpu.VMEM((4 * CH, dp), jnp.float32),
            pltpu.VMEM((CH, dp), jnp.float32),
            pltpu.SemaphoreType.DMA,
        ],
    )
    def pool_sc(x_hbm, a1, a2, a3, a4, out_hbm, sbuf, idxbuf, rowsbuf, outbuf, sem):
        wid = lax.axis_index("s") * NC + lax.axis_index("c")
        adjs = [a1, a2, a3, a4]
        for k in range(4):
            cid = wid + NW * k

            @pl.when(cid < NCH)
            def _():
                for deg in range(5):
                    base = deg * N_PER_DEG + cid * CH
                    pltpu.sync_copy(x_hbm.at[pl.ds(base, CH)], sbuf)
                    if deg == 0:
                        pltpu.sync_copy(sbuf, out_hbm.at[pl.ds(base, CH)])
                        continue
                    n = CH * deg
                    pltpu.sync_copy(adjs[deg - 1].at[pl.ds(cid * n, n)],
                                    idxbuf.at[pl.ds(0, n)])
                    pltpu.async_copy(x_hbm.at[idxbuf.at[pl.ds(0, n)]],
                                     rowsbuf.at[pl.ds(0, n)], sem).wait()

                    def row_body(i, _, deg=deg):
                        for v in range(nv):
                            acc = sbuf[i, pl.ds(v * 16, 16)]
                            for j in range(deg):
                                acc = jnp.maximum(acc, rowsbuf[i * deg + j, pl.ds(v * 16, 16)])
                            outbuf[i, pl.ds(v * 16, 16)] = acc
                        return 0

                    lax.fori_loop(0, CH, row_body, 0)
                    pltpu.sync_copy(outbuf, out_hbm.at[pl.ds(base, CH)])

    return pool_sc


def _dense_bn_body(x_ref, w_ref, b_ref, g_ref, beta_ref, o_ref):
    h = jnp.tanh(jnp.dot(x_ref[...], w_ref[...], preferred_element_type=jnp.float32)
                 + b_ref[...])
    o_ref[...] = g_ref[...] * h / jnp.sqrt(jnp.float32(1.0) + jnp.float32(1e-3)) + beta_ref[...]


def _dense_bn(x, w1p, b1, gamma, beta):
    bs = 2000
    return pl.pallas_call(
        _dense_bn_body,
        grid=(N_ATOMS // bs,),
        in_specs=[
            pl.BlockSpec((bs, 48), lambda i: (i, 0)),
            pl.BlockSpec((48, 256), lambda i: (0, 0)),
            pl.BlockSpec((1, 256), lambda i: (0, 0)),
            pl.BlockSpec((1, 256), lambda i: (0, 0)),
            pl.BlockSpec((1, 256), lambda i: (0, 0)),
        ],
        out_specs=pl.BlockSpec((bs, 256), lambda i: (i, 0)),
        out_shape=jax.ShapeDtypeStruct((N_ATOMS, 256), jnp.float32),
    )(x, w1p, b1, gamma, beta)


def _make_readout_sc():
    mesh = plsc.VectorSubcoreMesh(core_axis_name="c", subcore_axis_name="s")
    CR = 1000
    HALF = N_ATOMS // 2

    @functools.partial(
        pl.kernel,
        mesh=mesh,
        out_type=[
            jax.ShapeDtypeStruct((2, BATCH, 256), jnp.float32),
            jax.ShapeDtypeStruct((2, BATCH, 256), jnp.float32),
        ],
        compiler_params=pltpu.CompilerParams(use_tc_tiling_on_sc=False,
                                             needs_layout_passes=False),
        scratch_types=[
            pltpu.VMEM((CR, 16), jnp.float32),
            pltpu.VMEM((CR,), jnp.int32),
            pltpu.VMEM((BATCH, 16), jnp.float32),
            pltpu.VMEM((BATCH, 16), jnp.float32),
        ],
    )
    def readout_sc(y_hbm, memb_hbm, sum_hbm, max_hbm, ybuf, mbuf, accs, accm):
        wid = lax.axis_index("s") * NC + lax.axis_index("c")
        cg = wid % 16
        half = wid // 16
        lane = lax.iota(jnp.int32, 16)
        zero = jnp.zeros((16,), jnp.float32)
        ninf = jnp.full((16,), -jnp.inf, jnp.float32)

        def init_body(i, _):
            accs[i, :] = zero
            accm[i, :] = ninf
            return 0

        lax.fori_loop(0, BATCH, init_body, 0)

        def chunk_body(c, _):
            r0 = half * HALF + c * CR
            pltpu.sync_copy(y_hbm.at[pl.ds(r0, CR), pl.ds(cg * 16, 16)], ybuf)
            pltpu.sync_copy(memb_hbm.at[pl.ds(r0, CR)], mbuf)

            def row_body(i, _):
                seg = plsc.load_gather(mbuf, [jnp.full((16,), i, jnp.int32)])
                yv = ybuf[i, :]
                cs = plsc.load_gather(accs, [seg, lane])
                plsc.store_scatter(accs, [seg, lane], cs + yv)
                cm = plsc.load_gather(accm, [seg, lane])
                plsc.store_scatter(accm, [seg, lane], jnp.maximum(cm, yv))
                return 0

            lax.fori_loop(0, CR, row_body, 0)
            return 0

        lax.fori_loop(0, HALF // CR, chunk_body, 0)
        pltpu.sync_copy(accs, sum_hbm.at[half, :, pl.ds(cg * 16, 16)])
        pltpu.sync_copy(accm, max_hbm.at[half, :, pl.ds(cg * 16, 16)])

    return readout_sc


def _final_body(sp_ref, mp_ref, wa_ref, wb_ref, ba_ref, bb_ref, oa_ref, ob_ref):
    s = sp_ref[0] + sp_ref[1]
    m = jnp.maximum(mp_ref[0], mp_ref[1])
    g = jnp.tanh(jnp.concatenate([s, m], axis=1))
    a = jnp.dot(g, wa_ref[...], preferred_element_type=jnp.float32) + ba_ref[...]
    b = jnp.dot(g, wb_ref[...], preferred_element_type=jnp.float32) + bb_ref[...]
    mx = jnp.maximum(a, b)
    ea = jnp.exp(a - mx)
    eb = jnp.exp(b - mx)
    den = ea + eb
    oa_ref[...] = ea / den
    ob_ref[...] = eb / den


def _final(sumparts, maxparts, w2a, w2b, b2a, b2b):
    return pl.pallas_call(
        _final_body,
        out_shape=[
            jax.ShapeDtypeStruct((BATCH, NTASKS), jnp.float32),
            jax.ShapeDtypeStruct((BATCH, NTASKS), jnp.float32),
        ],
    )(sumparts, maxparts, w2a, w2b, b2a, b2b)


def kernel(atom_features, Wself0, Wrel0, bias0, Wself1, Wrel1, bias1,
           Wself2, Wrel2, bias2, Wself3, Wrel3, bias3, W1, b1, gamma, beta,
           W2, b2, degree_slice, membership,
           deg_adj_1, deg_adj_2, deg_adj_3, deg_adj_4):
    adj_flat = [jnp.reshape(a, (-1,)) for a in (deg_adj_1, deg_adj_2, deg_adj_3, deg_adj_4)]

    wselfs = [Wself0, Wself1, Wself2, Wself3]
    wrels = [Wrel0, Wrel1, Wrel2, Wrel3]
    biases = [bias0, bias1, bias2, bias3]
    din_ps = [128] + PADS[:-1]

    x = atom_features
    for l in range(4):
        din, dout = wselfs[l].shape[1], wselfs[l].shape[2]
        din_p, dp = din_ps[l], PADS[l]
        wself = jnp.pad(wselfs[l], ((0, 0), (0, din_p - din), (0, dp - dout)))
        wrel = jnp.pad(wrels[l], ((0, 0), (0, din_p - din), (0, dp - dout)))
        bias3d = jnp.pad(biases[l], ((0, 0), (0, dp - dout)))[:, None, :]
        s, p = _layer_matmuls(x, wself, wrel, bias3d, din_p, dp)
        x = _make_conv_sc(dp)(s, p[0], p[1], p[2], p[3], *adj_flat)

    x = _make_pool_sc(48)(x, *adj_flat)

    w1p = jnp.pad(W1, ((0, 48 - W1.shape[0]), (0, 0)))
    y = _dense_bn(x, w1p, b1[None, :], gamma[None, :], beta[None, :])

    sumparts, maxparts = _make_readout_sc()(y, membership)

    oa, ob = _final(sumparts, maxparts,
                    W2[:, 0::2], W2[:, 1::2], b2[None, 0::2], b2[None, 1::2])
    return jnp.stack([oa, ob], axis=-1)


# speedup vs baseline: 1.4042x; 1.2406x over previous
import functools
import math

import jax
import jax.numpy as jnp
from jax import lax
from jax.experimental import pallas as pl
from jax.experimental.pallas import tpu as pltpu
from jax.experimental.pallas import tpu_sc as plsc

N_PER_DEG = 20000
N_ATOMS = 5 * N_PER_DEG
BATCH = 512
NTASKS = 12
PADS = [16, 32, 32, 48]
NC, NS = 2, 16
NW = NC * NS
CH = 160
NCH = N_PER_DEG // CH

_SELU_ALPHA = 1.6732632423543772
_SELU_SCALE = 1.0507009873554805


def _selu(x):
    return _SELU_SCALE * jnp.where(x > 0, x, _SELU_ALPHA * (jnp.exp(x) - 1.0))


def _layer_mm_body(x_ref, wself_ref, wrel_ref, bias_ref, s_ref, p_ref):
    x = x_ref[...]
    s_ref[...] = jnp.dot(x, wself_ref[0], preferred_element_type=jnp.float32) + bias_ref[0]
    for d in range(4):
        p_ref[d] = jnp.dot(x, wrel_ref[d], preferred_element_type=jnp.float32)


def _layer_matmuls(x, wself, wrel, bias3d, din_p, dp):
    bs = 2000
    nblk = N_ATOMS // bs
    per_bucket = N_PER_DEG // bs
    return pl.pallas_call(
        _layer_mm_body,
        grid=(nblk,),
        in_specs=[
            pl.BlockSpec((bs, din_p), lambda i: (i, 0)),
            pl.BlockSpec((1, din_p, dp), lambda i: (i // per_bucket, 0, 0)),
            pl.BlockSpec((4, din_p, dp), lambda i: (0, 0, 0)),
            pl.BlockSpec((1, 1, dp), lambda i: (i // per_bucket, 0, 0)),
        ],
        out_specs=[
            pl.BlockSpec((bs, dp), lambda i: (i, 0)),
            pl.BlockSpec((4, bs, dp), lambda i: (0, i, 0)),
        ],
        out_shape=[
            jax.ShapeDtypeStruct((N_ATOMS, dp), jnp.float32),
            jax.ShapeDtypeStruct((4, N_ATOMS, dp), jnp.float32),
        ],
    )(x, wself, wrel, bias3d)


def _make_conv_sc(dp):
    nv = dp // 16
    mesh = plsc.VectorSubcoreMesh(core_axis_name="c", subcore_axis_name="s")

    @functools.partial(
        pl.kernel,
        mesh=mesh,
        out_type=jax.ShapeDtypeStruct((N_ATOMS, dp), jnp.float32),
        compiler_params=pltpu.CompilerParams(use_tc_tiling_on_sc=False),
        scratch_types=[
            pltpu.VMEM((CH, dp), jnp.float32),
            pltpu.VMEM((4 * CH,), jnp.int32),
            pltpu.VMEM((4 * CH, dp), jnp.float32),
            pltpu.VMEM((CH, dp), jnp.float32),
            pltpu.SemaphoreType.DMA,
        ],
    )
    def conv_sc(s_hbm, p1, p2, p3, p4, a1, a2, a3, a4, out_hbm,
                sbuf, idxbuf, rowsbuf, outbuf, sem):
        wid = lax.axis_index("s") * NC + lax.axis_index("c")
        ptabs = [p1, p2, p3, p4]
        adjs = [a1, a2, a3, a4]
        for k in range(4):
            cid = wid + NW * k

            @pl.when(cid < NCH)
            def _():
                for deg in range(5):
                    base = deg * N_PER_DEG + cid * CH
                    pltpu.sync_copy(s_hbm.at[pl.ds(base, CH)], sbuf)
                    if deg > 0:
                        n = CH * deg
                        pltpu.sync_copy(adjs[deg - 1].at[pl.ds(cid * n, n)],
                                        idxbuf.at[pl.ds(0, n)])
                        pltpu.async_copy(ptabs[deg - 1].at[idxbuf.at[pl.ds(0, n)]],
                                         rowsbuf.at[pl.ds(0, n)], sem).wait()

                    def row_body(i, _, deg=deg):
                        for v in range(nv):
                            acc = sbuf[i, pl.ds(v * 16, 16)]
                            for j in range(deg):
                                acc = acc + rowsbuf[i * deg + j, pl.ds(v * 16, 16)]
                            outbuf[i, pl.ds(v * 16, 16)] = _selu(acc)
                        return 0

                    lax.fori_loop(0, CH, row_body, 0)
                    pltpu.sync_copy(outbuf, out_hbm.at[pl.ds(base, CH)])

    return conv_sc


def _make_pool_sc(dp):
    nv = dp // 16
    mesh = plsc.VectorSubcoreMesh(core_axis_name="c", subcore_axis_name="s")

    @functools.partial(
        pl.kernel,
        mesh=mesh,
        out_type=jax.ShapeDtypeStruct((N_ATOMS, dp), jnp.float32),
        compiler_params=pltpu.CompilerParams(use_tc_tiling_on_sc=False),
        scratch_types=[
            pltpu.VMEM((CH, dp), jnp.float32),
            pltpu.VMEM((4 * CH,), jnp.int32),
            pltpu.VMEM((4 * CH, dp), jnp.float32),
            pltpu.VMEM((CH, dp), jnp.float32),
            pltpu.SemaphoreType.DMA,
        ],
    )
    def pool_sc(x_hbm, a1, a2, a3, a4, out_hbm, sbuf, idxbuf, rowsbuf, outbuf, sem):
        wid = lax.axis_index("s") * NC + lax.axis_index("c")
        adjs = [a1, a2, a3, a4]
        for k in range(4):
            cid = wid + NW * k

            @pl.when(cid < NCH)
            def _():
                for deg in range(5):
                    base = deg * N_PER_DEG + cid * CH
                    pltpu.sync_copy(x_hbm.at[pl.ds(base, CH)], sbuf)
                    if deg == 0:
                        pltpu.sync_copy(sbuf, out_hbm.at[pl.ds(base, CH)])
                        continue
                    n = CH * deg
                    pltpu.sync_copy(adjs[deg - 1].at[pl.ds(cid * n, n)],
                                    idxbuf.at[pl.ds(0, n)])
                    pltpu.async_copy(x_hbm.at[idxbuf.at[pl.ds(0, n)]],
                                     rowsbuf.at[pl.ds(0, n)], sem).wait()

                    def row_body(i, _, deg=deg):
                        for v in range(nv):
                            acc = sbuf[i, pl.ds(v * 16, 16)]
                            for j in range(deg):
                                acc = jnp.maximum(acc, rowsbuf[i * deg + j, pl.ds(v * 16, 16)])
                            outbuf[i, pl.ds(v * 16, 16)] = acc
                        return 0

                    lax.fori_loop(0, CH, row_body, 0)
                    pltpu.sync_copy(outbuf, out_hbm.at[pl.ds(base, CH)])

    return pool_sc


def _dense_bn_body(x_ref, w_ref, b_ref, g_ref, beta_ref, o_ref):
    h = jnp.tanh(jnp.dot(x_ref[...], w_ref[...], preferred_element_type=jnp.float32)
                 + b_ref[...])
    o_ref[...] = g_ref[...] * h / jnp.sqrt(jnp.float32(1.0) + jnp.float32(1e-3)) + beta_ref[...]


def _dense_bn(x, w1p, b1, gamma, beta):
    bs = 2000
    return pl.pallas_call(
        _dense_bn_body,
        grid=(N_ATOMS // bs,),
        in_specs=[
            pl.BlockSpec((bs, 48), lambda i: (i, 0)),
            pl.BlockSpec((48, 256), lambda i: (0, 0)),
            pl.BlockSpec((1, 256), lambda i: (0, 0)),
            pl.BlockSpec((1, 256), lambda i: (0, 0)),
            pl.BlockSpec((1, 256), lambda i: (0, 0)),
        ],
        out_specs=pl.BlockSpec((bs, 256), lambda i: (i, 0)),
        out_shape=jax.ShapeDtypeStruct((N_ATOMS, 256), jnp.float32),
    )(x, w1p, b1, gamma, beta)


def _make_readout_sc():
    mesh = plsc.VectorSubcoreMesh(core_axis_name="c", subcore_axis_name="s")
    CR = 2000
    HALF = N_ATOMS // 2

    @functools.partial(
        pl.kernel,
        mesh=mesh,
        out_type=[
            jax.ShapeDtypeStruct((2, BATCH, 256), jnp.float32),
            jax.ShapeDtypeStruct((2, BATCH, 256), jnp.float32),
        ],
        compiler_params=pltpu.CompilerParams(use_tc_tiling_on_sc=False,
                                             needs_layout_passes=False),
        scratch_types=[
            pltpu.VMEM((CR, 16), jnp.float32),
            pltpu.VMEM((CR,), jnp.int32),
            pltpu.VMEM((BATCH, 16), jnp.float32),
            pltpu.VMEM((BATCH, 16), jnp.float32),
        ],
    )
    def readout_sc(y_hbm, memb_hbm, sum_hbm, max_hbm, ybuf, mbuf, accs, accm):
        wid = lax.axis_index("s") * NC + lax.axis_index("c")
        cg = wid % 16
        half = wid // 16
        lane = lax.iota(jnp.int32, 16)
        zero = jnp.zeros((16,), jnp.float32)
        ninf = jnp.full((16,), -jnp.inf, jnp.float32)

        def init_body(i, _):
            accs[i, :] = zero
            accm[i, :] = ninf
            return 0

        lax.fori_loop(0, BATCH, init_body, 0)

        def chunk_body(c, _):
            r0 = half * HALF + c * CR
            pltpu.sync_copy(y_hbm.at[pl.ds(r0, CR), pl.ds(cg * 16, 16)], ybuf)
            pltpu.sync_copy(memb_hbm.at[pl.ds(r0, CR)], mbuf)

            def blk_body(b, _):
                i0 = b * 16
                mvec = mbuf[pl.ds(i0, 16)]
                seg0 = plsc.load_gather(mbuf, [jnp.full((16,), i0, jnp.int32)])
                uniform = jnp.all(mvec == seg0)

                def fast(_):
                    rows = [ybuf[i0 + j, :] for j in range(16)]
                    ss, mm = list(rows), list(rows)
                    w = 16
                    while w > 1:
                        w //= 2
                        ss = [ss[j] + ss[j + w] for j in range(w)]
                        mm = [jnp.maximum(mm[j], mm[j + w]) for j in range(w)]
                    cs = plsc.load_gather(accs, [seg0, lane])
                    plsc.store_scatter(accs, [seg0, lane], cs + ss[0])
                    cm = plsc.load_gather(accm, [seg0, lane])
                    plsc.store_scatter(accm, [seg0, lane], jnp.maximum(cm, mm[0]))
                    return 0

                def slow(_):
                    def row_body(i, _):
                        seg = plsc.load_gather(mbuf, [jnp.full((16,), i, jnp.int32)])
                        yv = ybuf[i, :]
                        cs = plsc.load_gather(accs, [seg, lane])
                        plsc.store_scatter(accs, [seg, lane], cs + yv)
                        cm = plsc.load_gather(accm, [seg, lane])
                        plsc.store_scatter(accm, [seg, lane], jnp.maximum(cm, yv))
                        return 0

                    return lax.fori_loop(i0, i0 + 16, row_body, 0)

                lax.cond(uniform, fast, slow, 0)
                return 0

            lax.fori_loop(0, CR // 16, blk_body, 0)
            return 0

        lax.fori_loop(0, HALF // CR, chunk_body, 0)
        pltpu.sync_copy(accs, sum_hbm.at[half, :, pl.ds(cg * 16, 16)])
        pltpu.sync_copy(accm, max_hbm.at[half, :, pl.ds(cg * 16, 16)])

    return readout_sc


def _final_body(sp_ref, mp_ref, wa_ref, wb_ref, ba_ref, bb_ref, oa_ref, ob_ref):
    s = sp_ref[0] + sp_ref[1]
    m = jnp.maximum(mp_ref[0], mp_ref[1])
    g = jnp.tanh(jnp.concatenate([s, m], axis=1))
    a = jnp.dot(g, wa_ref[...], preferred_element_type=jnp.float32) + ba_ref[...]
    b = jnp.dot(g, wb_ref[...], preferred_element_type=jnp.float32) + bb_ref[...]
    mx = jnp.maximum(a, b)
    ea = jnp.exp(a - mx)
    eb = jnp.exp(b - mx)
    den = ea + eb
    oa_ref[...] = ea / den
    ob_ref[...] = eb / den


def _final(sumparts, maxparts, w2a, w2b, b2a, b2b):
    return pl.pallas_call(
        _final_body,
        out_shape=[
            jax.ShapeDtypeStruct((BATCH, NTASKS), jnp.float32),
            jax.ShapeDtypeStruct((BATCH, NTASKS), jnp.float32),
        ],
    )(sumparts, maxparts, w2a, w2b, b2a, b2b)


def kernel(atom_features, Wself0, Wrel0, bias0, Wself1, Wrel1, bias1,
           Wself2, Wrel2, bias2, Wself3, Wrel3, bias3, W1, b1, gamma, beta,
           W2, b2, degree_slice, membership,
           deg_adj_1, deg_adj_2, deg_adj_3, deg_adj_4):
    adj_flat = [jnp.reshape(a, (-1,)) for a in (deg_adj_1, deg_adj_2, deg_adj_3, deg_adj_4)]

    wselfs = [Wself0, Wself1, Wself2, Wself3]
    wrels = [Wrel0, Wrel1, Wrel2, Wrel3]
    biases = [bias0, bias1, bias2, bias3]
    din_ps = [128] + PADS[:-1]

    x = atom_features
    for l in range(4):
        din, dout = wselfs[l].shape[1], wselfs[l].shape[2]
        din_p, dp = din_ps[l], PADS[l]
        wself = jnp.pad(wselfs[l], ((0, 0), (0, din_p - din), (0, dp - dout)))
        wrel = jnp.pad(wrels[l], ((0, 0), (0, din_p - din), (0, dp - dout)))
        bias3d = jnp.pad(biases[l], ((0, 0), (0, dp - dout)))[:, None, :]
        s, p = _layer_matmuls(x, wself, wrel, bias3d, din_p, dp)
        x = _make_conv_sc(dp)(s, p[0], p[1], p[2], p[3], *adj_flat)

    x = _make_pool_sc(48)(x, *adj_flat)

    w1p = jnp.pad(W1, ((0, 48 - W1.shape[0]), (0, 0)))
    y = _dense_bn(x, w1p, b1[None, :], gamma[None, :], beta[None, :])

    sumparts, maxparts = _make_readout_sc()(y, membership)

    oa, ob = _final(sumparts, maxparts,
                    W2[:, 0::2], W2[:, 1::2], b2[None, 0::2], b2[None, 1::2])
    return jnp.stack([oa, ob], axis=-1)
